# baseline (speedup 1.0000x reference)
import functools

import jax
import jax.numpy as jnp
from jax import lax
from jax.experimental import pallas as pl
from jax.experimental.pallas import tpu as pltpu
from jax.experimental.pallas import tpu_sc as plsc

VOCAB = 100000
EMBED_DIM = 64
BATCH = 4096
HIST = 50

NUM_CORES = 2
NUM_SUBCORES = 16
NW = NUM_CORES * NUM_SUBCORES
B_PER_W = BATCH // NW
LANES = 16
J = EMBED_DIM // LANES
PADB = B_PER_W + 1


def _make_gather():
    mesh = plsc.VectorSubcoreMesh(core_axis_name="c", subcore_axis_name="s")

    @functools.partial(
        pl.kernel,
        mesh=mesh,
        out_type=jax.ShapeDtypeStruct((HIST, 8, NW, 8, B_PER_W), jnp.float32),
        scratch_types=[
            pltpu.VMEM((HIST, B_PER_W), jnp.int32),
            pltpu.VMEM((B_PER_W, 128), jnp.float32),
            pltpu.VMEM((B_PER_W, 128), jnp.float32),
            pltpu.VMEM((B_PER_W, 128), jnp.float32),
            pltpu.VMEM((B_PER_W, 128), jnp.float32),
            pltpu.VMEM((8, 8, PADB), jnp.float32),
            pltpu.VMEM((8, 8, PADB), jnp.float32),
            pltpu.SemaphoreType.DMA,
            pltpu.SemaphoreType.DMA,
            pltpu.SemaphoreType.DMA,
            pltpu.SemaphoreType.DMA,
            pltpu.SemaphoreType.DMA,
            pltpu.SemaphoreType.DMA,
        ],
        compiler_params=pltpu.CompilerParams(
            use_tc_tiling_on_sc=False, needs_layout_passes=False,
            disable_bounds_checks=True),
    )
    def gather_kernel(idx_hbm, table_hbm, out_hbm, idx_v, g0, g1, g2, g3,
                      t0, t1, sg0, sg1, sg2, sg3, sw0, sw1):
        wid = lax.axis_index("s") * NUM_CORES + lax.axis_index("c")
        bcol = wid * B_PER_W
        pltpu.sync_copy(idx_hbm.at[:, pl.ds(bcol, B_PER_W)], idx_v)

        iota = lax.iota(jnp.int32, LANES)
        e_hi = [(iota + LANES * j) >> 3 for j in range(J)]
        e_lo = [(iota + LANES * j) & 7 for j in range(J)]
        zeros = jnp.zeros((LANES,), jnp.int32)
        ones = jnp.full((LANES,), 1, jnp.int32)

        def fire_gather(h, buf, sg):
            pltpu.async_copy(table_hbm.at[idx_v.at[h]], buf, sg)

        def wait_gather(buf, sg):
            pltpu.make_async_copy(table_hbm.at[idx_v.at[0]], buf, sg).wait()

        def fire_write(h, tbuf, sw):
            pltpu.async_copy(tbuf.at[:, :, pl.ds(0, B_PER_W)],
                             out_hbm.at[h, :, wid], sw)

        def drain_write(tbuf, sw):
            pltpu.make_async_copy(tbuf.at[:, :, pl.ds(0, B_PER_W)],
                                  out_hbm.at[0, :, wid], sw).wait()

        def transpose(gbuf, tbuf):
            @plsc.parallel_loop(0, B_PER_W, step=1, unroll=8)
            def _(b):
                bvec = zeros + b
                for j in range(J):
                    v = gbuf[b, pl.ds(LANES * j, LANES)]
                    plsc.store_scatter(tbuf, [e_hi[j], e_lo[j], bvec], v)

        gb = [(g0, sg0), (g1, sg1), (g2, sg2), (g3, sg3)]
        tbufs = [(t0, sw0), (t1, sw1)]

        def step(i, h, p4, p2, always_drain):
            gbuf, sg = gb[p4]
            tbuf, sw = tbufs[p2]
            wait_gather(gbuf, sg)

            if always_drain:
                drain_write(tbuf, sw)
            else:
                @pl.when(i > 0)
                def _():
                    drain_write(tbuf, sw)

            transpose(gbuf, tbuf)

            @pl.when(h + 4 < HIST)
            def _():
                fire_gather(h + 4, gbuf, sg)

            fire_write(h, tbuf, sw)

        for k in range(4):
            fire_gather(k, *gb[k])

        def body(i, carry):
            for k in range(4):
                step(i, 4 * i + k, k, k % 2, k >= 2)
            return carry

        lax.fori_loop(0, (HIST - 2) // 4, body, 0)
        for h, p4, p2 in ((HIST - 2, 0, 0), (HIST - 1, 1, 1)):
            gbuf, sg = gb[p4]
            tbuf, sw = tbufs[p2]
            wait_gather(gbuf, sg)
            drain_write(tbuf, sw)
            transpose(gbuf, tbuf)
            fire_write(h, tbuf, sw)
        drain_write(t0, sw0)
        drain_write(t1, sw1)

    return gather_kernel


_gather = _make_gather()


def kernel(x, table):
    idx = x.T.astype(jnp.int32)
    table_p = jnp.pad(table, ((0, 0), (0, 128 - EMBED_DIM)))
    out5 = _gather(idx, table_p)
    return out5.transpose(2, 4, 0, 1, 3).reshape(BATCH, HIST, EMBED_DIM)


# speedup vs baseline: 1.0939x; 1.0939x over previous
import functools

import jax
import jax.numpy as jnp
from jax import lax
from jax.experimental import pallas as pl
from jax.experimental.pallas import tpu as pltpu
from jax.experimental.pallas import tpu_sc as plsc

VOCAB = 100000
EMBED_DIM = 64
BATCH = 4096
HIST = 50

NUM_CORES = 2
NUM_SUBCORES = 16
NW = NUM_CORES * NUM_SUBCORES
B_PER_W = BATCH // NW
LANES = 16
J = EMBED_DIM // LANES
PADB = B_PER_W + 1


def _make_gather():
    mesh = plsc.VectorSubcoreMesh(core_axis_name="c", subcore_axis_name="s")

    @functools.partial(
        pl.kernel,
        mesh=mesh,
        out_type=jax.ShapeDtypeStruct((HIST, 8, NW, 8, B_PER_W), jnp.float32),
        scratch_types=[
            pltpu.VMEM((HIST, B_PER_W), jnp.int32),
            pltpu.VMEM((B_PER_W, EMBED_DIM), jnp.float32),
            pltpu.VMEM((B_PER_W, EMBED_DIM), jnp.float32),
            pltpu.VMEM((B_PER_W, EMBED_DIM), jnp.float32),
            pltpu.VMEM((B_PER_W, EMBED_DIM), jnp.float32),
            pltpu.VMEM((8, 8, PADB), jnp.float32),
            pltpu.VMEM((8, 8, PADB), jnp.float32),
            pltpu.SemaphoreType.DMA,
            pltpu.SemaphoreType.DMA,
            pltpu.SemaphoreType.DMA,
            pltpu.SemaphoreType.DMA,
            pltpu.SemaphoreType.DMA,
            pltpu.SemaphoreType.DMA,
        ],
        compiler_params=pltpu.CompilerParams(
            use_tc_tiling_on_sc=False, needs_layout_passes=False,
            disable_bounds_checks=True),
    )
    def gather_kernel(idx_hbm, table_hbm, out_hbm, idx_v, g0, g1, g2, g3,
                      t0, t1, sg0, sg1, sg2, sg3, sw0, sw1):
        wid = lax.axis_index("s") * NUM_CORES + lax.axis_index("c")
        bcol = wid * B_PER_W
        pltpu.sync_copy(idx_hbm.at[:, pl.ds(bcol, B_PER_W)], idx_v)

        iota = lax.iota(jnp.int32, LANES)
        e_hi = [(iota + LANES * j) >> 3 for j in range(J)]
        e_lo = [(iota + LANES * j) & 7 for j in range(J)]
        zeros = jnp.zeros((LANES,), jnp.int32)
        ones = jnp.full((LANES,), 1, jnp.int32)

        def fire_gather(h, buf, sg):
            pltpu.async_copy(table_hbm.at[idx_v.at[h]], buf, sg)

        def wait_gather(buf, sg):
            pltpu.make_async_copy(table_hbm.at[idx_v.at[0]], buf, sg).wait()

        def fire_write(h, tbuf, sw):
            pltpu.async_copy(tbuf.at[:, :, pl.ds(0, B_PER_W)],
                             out_hbm.at[h, :, wid], sw)

        def drain_write(tbuf, sw):
            pltpu.make_async_copy(tbuf.at[:, :, pl.ds(0, B_PER_W)],
                                  out_hbm.at[0, :, wid], sw).wait()

        def transpose(gbuf, tbuf):
            @plsc.parallel_loop(0, B_PER_W, step=1, unroll=8)
            def _(b):
                bvec = zeros + b
                for j in range(J):
                    v = gbuf[b, pl.ds(LANES * j, LANES)]
                    plsc.store_scatter(tbuf, [e_hi[j], e_lo[j], bvec], v)

        gb = [(g0, sg0), (g1, sg1), (g2, sg2), (g3, sg3)]
        tbufs = [(t0, sw0), (t1, sw1)]

        def step(i, h, p4, p2, always_drain):
            gbuf, sg = gb[p4]
            tbuf, sw = tbufs[p2]
            wait_gather(gbuf, sg)

            if always_drain:
                drain_write(tbuf, sw)
            else:
                @pl.when(i > 0)
                def _():
                    drain_write(tbuf, sw)

            transpose(gbuf, tbuf)

            @pl.when(h + 4 < HIST)
            def _():
                fire_gather(h + 4, gbuf, sg)

            fire_write(h, tbuf, sw)

        for k in range(4):
            fire_gather(k, *gb[k])

        def body(i, carry):
            for k in range(4):
                step(i, 4 * i + k, k, k % 2, k >= 2)
            return carry

        lax.fori_loop(0, (HIST - 2) // 4, body, 0)
        for h, p4, p2 in ((HIST - 2, 0, 0), (HIST - 1, 1, 1)):
            gbuf, sg = gb[p4]
            tbuf, sw = tbufs[p2]
            wait_gather(gbuf, sg)
            drain_write(tbuf, sw)
            transpose(gbuf, tbuf)
            fire_write(h, tbuf, sw)
        drain_write(t0, sw0)
        drain_write(t1, sw1)

    return gather_kernel


_gather = _make_gather()


def kernel(x, table):
    idx = x.T.astype(jnp.int32)
    out5 = _gather(idx, table)
    return out5.transpose(2, 4, 0, 1, 3).reshape(BATCH, HIST, EMBED_DIM)
